# Initial kernel scaffold; baseline (speedup 1.0000x reference)
import jax
import jax.numpy as jnp
from jax.experimental import pallas as pl


def kernel(inputs, mask, W):
    raise NotImplementedError("write your pallas kernel here")


# speedup vs baseline: 3.7109x; 3.7109x over previous
import functools

import jax
import jax.numpy as jnp
from jax.experimental import pallas as pl
from jax.experimental.pallas import tpu as pltpu

B, S, D = 16, 4096, 128
SBLK = 512


def _mm_kernel(x_ref, m_ref, w_ref, o_ref, fin_ref, ri_ref, nv_ref,
               lens_ref, cand_ref):
    k = pl.program_id(0)
    nsteps = pl.num_programs(0)

    @pl.when(k == 0)
    def _init():
        lens_ref[...] = jnp.zeros_like(lens_ref)
        cand_ref[...] = jnp.zeros_like(cand_ref)

    x = x_ref[...]
    m = m_ref[...]
    w = w_ref[...]
    y2 = jnp.dot(x.reshape(B * SBLK, D), w,
                 preferred_element_type=jnp.float32)
    y = y2.reshape(B, SBLK, D)
    o_ref[...] = y * m[:, :, None]

    local_len = jnp.sum(m, axis=1)
    lens_ref[...] = lens_ref[...] + local_len[None, :]
    m_next = jnp.concatenate([m[:, 1:], jnp.zeros((B, 1), m.dtype)], axis=1)
    flag = m * (1.0 - m_next)
    contrib = jnp.sum(flag[:, :, None] * y, axis=1)
    has = jnp.sum(flag, axis=1)[:, None] > 0.0
    cand_ref[...] = jnp.where(has, contrib, cand_ref[...])

    @pl.when(k == nsteps - 1)
    def _finish():
        lens = lens_ref[...]
        iota = jax.lax.broadcasted_iota(jnp.int32, (1, B), 1)
        rank = jnp.zeros((1, B), jnp.int32)
        for j in range(B):
            lj = lens[0, j]
            gt = (lj > lens).astype(jnp.int32)
            tie = jnp.logical_and(lj == lens, iota > j).astype(jnp.int32)
            rank = rank + gt + tie
        ri_ref[...] = rank
        nv_ref[...] = jnp.sum((lens >= 1.0).astype(jnp.int32),
                              axis=1, keepdims=True)
        rows = jax.lax.broadcasted_iota(jnp.int32, (B, B), 0)
        p = (rows == jnp.broadcast_to(rank, (B, B))).astype(jnp.float32)
        fin_ref[...] = jnp.dot(p, cand_ref[...],
                               preferred_element_type=jnp.float32)


@jax.jit
def kernel(inputs, mask, W):
    grid = S // SBLK
    restored, fin, ri, nv = pl.pallas_call(
        _mm_kernel,
        grid=(grid,),
        in_specs=[
            pl.BlockSpec((B, SBLK, D), lambda k: (0, k, 0)),
            pl.BlockSpec((B, SBLK), lambda k: (0, k)),
            pl.BlockSpec((D, D), lambda k: (0, 0)),
        ],
        out_specs=[
            pl.BlockSpec((B, SBLK, D), lambda k: (0, k, 0)),
            pl.BlockSpec((B, D), lambda k: (0, 0)),
            pl.BlockSpec((1, B), lambda k: (0, 0)),
            pl.BlockSpec((1, 1), lambda k: (0, 0)),
        ],
        out_shape=[
            jax.ShapeDtypeStruct((B, S, D), jnp.float32),
            jax.ShapeDtypeStruct((B, D), jnp.float32),
            jax.ShapeDtypeStruct((1, B), jnp.int32),
            jax.ShapeDtypeStruct((1, 1), jnp.int32),
        ],
        scratch_shapes=[
            pltpu.VMEM((1, B), jnp.float32),
            pltpu.VMEM((B, D), jnp.float32),
        ],
    )(inputs, mask, W)
    final_states = fin[None, :, :]
    restoration_indices = ri[0]
    num_valid = nv[0, 0]
    return (restored, final_states, restoration_indices, num_valid)


# speedup vs baseline: 3.7881x; 1.0208x over previous
import functools

import jax
import jax.numpy as jnp
from jax.experimental import pallas as pl
from jax.experimental.pallas import tpu as pltpu

B, S, D = 16, 4096, 128
SBLK = 1024


def _mm_kernel(x_ref, m_ref, w_ref, o_ref, fin_ref, ri_ref, nv_ref,
               lens_ref, cand_ref):
    k = pl.program_id(0)
    nsteps = pl.num_programs(0)

    @pl.when(k == 0)
    def _init():
        lens_ref[...] = jnp.zeros_like(lens_ref)
        cand_ref[...] = jnp.zeros_like(cand_ref)

    x = x_ref[...]
    m = m_ref[...]
    w = w_ref[...]
    y2 = jnp.dot(x.reshape(B * SBLK, D), w,
                 preferred_element_type=jnp.float32)
    y = y2.reshape(B, SBLK, D)
    o_ref[...] = y * m[:, :, None]

    local_len = jnp.sum(m, axis=1)
    lens_ref[...] = lens_ref[...] + local_len[None, :]
    m_next = jnp.concatenate([m[:, 1:], jnp.zeros((B, 1), m.dtype)], axis=1)
    flag = m * (1.0 - m_next)
    contrib = jnp.sum(flag[:, :, None] * y, axis=1)
    has = jnp.sum(flag, axis=1)[:, None] > 0.0
    cand_ref[...] = jnp.where(has, contrib, cand_ref[...])

    @pl.when(k == nsteps - 1)
    def _finish():
        lens = lens_ref[...]
        iota = jax.lax.broadcasted_iota(jnp.int32, (1, B), 1)
        rank = jnp.zeros((1, B), jnp.int32)
        for j in range(B):
            lj = lens[0, j]
            gt = (lj > lens).astype(jnp.int32)
            tie = jnp.logical_and(lj == lens, iota > j).astype(jnp.int32)
            rank = rank + gt + tie
        ri_ref[...] = rank
        nv_ref[...] = jnp.sum((lens >= 1.0).astype(jnp.int32),
                              axis=1, keepdims=True)
        rows = jax.lax.broadcasted_iota(jnp.int32, (B, B), 0)
        p = (rows == jnp.broadcast_to(rank, (B, B))).astype(jnp.float32)
        fin_ref[...] = jnp.dot(p, cand_ref[...],
                               preferred_element_type=jnp.float32)


@jax.jit
def kernel(inputs, mask, W):
    grid = S // SBLK
    restored, fin, ri, nv = pl.pallas_call(
        _mm_kernel,
        grid=(grid,),
        in_specs=[
            pl.BlockSpec((B, SBLK, D), lambda k: (0, k, 0)),
            pl.BlockSpec((B, SBLK), lambda k: (0, k)),
            pl.BlockSpec((D, D), lambda k: (0, 0)),
        ],
        out_specs=[
            pl.BlockSpec((B, SBLK, D), lambda k: (0, k, 0)),
            pl.BlockSpec((B, D), lambda k: (0, 0)),
            pl.BlockSpec((1, B), lambda k: (0, 0)),
            pl.BlockSpec((1, 1), lambda k: (0, 0)),
        ],
        out_shape=[
            jax.ShapeDtypeStruct((B, S, D), jnp.float32),
            jax.ShapeDtypeStruct((B, D), jnp.float32),
            jax.ShapeDtypeStruct((1, B), jnp.int32),
            jax.ShapeDtypeStruct((1, 1), jnp.int32),
        ],
        scratch_shapes=[
            pltpu.VMEM((1, B), jnp.float32),
            pltpu.VMEM((B, D), jnp.float32),
        ],
    )(inputs, mask, W)
    final_states = fin[None, :, :]
    restoration_indices = ri[0]
    num_valid = nv[0, 0]
    return (restored, final_states, restoration_indices, num_valid)
